# baseline (speedup 1.0000x reference)
import jax
import jax.numpy as jnp
from jax.experimental import pallas as pl

_NU, _NI, _NC, _D, _L = 50000, 20000, 1000, 64, 2


def _spmm(rows, cols, vals, x, n):
    return jax.ops.segment_sum(vals[:, None] * x[cols], rows, num_segments=n)


def _prop(rows, cols, vals, ego, n):
    allv = [ego]
    h = ego
    for _ in range(_L):
        h = _spmm(rows, cols, vals, h, n)
        allv.append(h)
    return jnp.stack(allv, axis=1).mean(axis=1)


def _l2n(x):
    return x / jnp.maximum(jnp.linalg.norm(x, axis=1, keepdims=True), 1e-12)


def _sup_body(u_ref, p_ref, n_ref, o_ref):
    u = u_ref[...]
    o_ref[...] = (jnp.sum(u * p_ref[...], axis=1, keepdims=True)
                  - jnp.sum(u * n_ref[...], axis=1, keepdims=True))


def kernel(users, pos_items, neg_items, users1, cates1, items2, cates2,
           user_emb, item_emb, cate_emb,
           rows_ui, cols_ui, vals_ui,
           rows_uc, cols_uc, vals_uc,
           rows_ic, cols_ic, vals_ic):
    e1 = _prop(rows_ui, cols_ui, vals_ui, jnp.concatenate([user_emb, item_emb], 0), _NU + _NI)
    ue1, ie1 = e1[:_NU], e1[_NU:]
    e2 = _prop(rows_uc, cols_uc, vals_uc, jnp.concatenate([user_emb, cate_emb], 0), _NU + _NC)
    ue2, ce1 = e2[:_NU], e2[_NU:]
    e3 = _prop(rows_ic, cols_ic, vals_ic, jnp.concatenate([item_emb, cate_emb], 0), _NI + _NC)
    ie2, ce2 = e3[:_NI], e3[_NI:]

    u = ue1[users]
    p = ie1[pos_items]
    n = ie1[neg_items]
    sup = pl.pallas_call(
        _sup_body,
        out_shape=jax.ShapeDtypeStruct((u.shape[0], 1), jnp.float32),
    )(u, p, n)[:, 0]

    un = _l2n(ue2)[users1]
    cn1 = _l2n(ce1)[cates1]
    pos_u = jnp.sum(un * cn1, -1)
    con_u = un @ cn1.T - pos_u[:, None]

    im = _l2n(ie2)[items2]
    cn2 = _l2n(ce2)[cates2]
    pos_i = jnp.sum(im * cn2, -1)
    con_i = im @ cn2.T - pos_i[:, None]

    return (sup, con_u, con_i)


# speedup vs baseline: 2.3033x; 2.3033x over previous
import functools

import jax
import jax.numpy as jnp
from jax import lax
from jax.experimental import pallas as pl
from jax.experimental.pallas import tpu as pltpu
from jax.experimental.pallas import tpu_sc as plsc

NU, NI, NCAT, D = 50000, 20000, 1000, 64
NS, LANES = 16, 16
CH = 256
SUB = 128
NSUB = CH // SUB


def _round_up(x, m):
    return (x + m - 1) // m * m


def _make_spmm(n_src, n_dst, e_half):
    n_src_pad = _round_up(n_src, 256)
    n_dst_pad = _round_up(n_dst, 256)
    e_pad = _round_up(e_half, NS * CH)
    nchunks = e_pad // (NS * CH)
    r0 = n_src_pad // 2
    r1 = n_dst_pad // 2
    n_pad = n_src_pad + n_dst_pad
    acc_rows = max(r0, r1) + 8
    passes = ((r0, 0, 0), (r1, n_src, n_src_pad))

    mesh = plsc.VectorSubcoreMesh(core_axis_name="c", subcore_axis_name="s")

    @functools.partial(
        pl.kernel,
        out_type=jax.ShapeDtypeStruct((n_pad, D), jnp.float32),
        mesh=mesh,
        compiler_params=pltpu.CompilerParams(use_tc_tiling_on_sc=False),
        scratch_types=[
            pltpu.VMEM_SHARED((acc_rows, D), jnp.float32),
            pltpu.VMEM((CH,), jnp.int32),
            pltpu.VMEM((CH,), jnp.int32),
            pltpu.VMEM((CH,), jnp.float32),
            pltpu.VMEM((NSUB, SUB), jnp.int32),
            pltpu.VMEM((CH, D), jnp.float32),
            pltpu.VMEM((64, D), jnp.float32),
            pltpu.SemaphoreType.DMA,
        ],
    )
    def spmm(rows_h, cols_h, vals_h, x_h, out_h,
             acc, rows_v, cols_v, vals_v, sidx, gbuf, zbuf, sem):
        c = lax.axis_index("c")
        s = lax.axis_index("s")

        zero16 = jnp.zeros((LANES,), jnp.float32)

        @pl.loop(0, 64)
        def _(i):
            for j in range(D // LANES):
                zbuf[i, pl.ds(j * LANES, LANES)] = zero16

        for p, (r, base0, out0) in enumerate(passes):
            base = base0 + c * r
            out_off = out0 + c * r
            nrt = r // NS
            full, tail = nrt // 64, nrt % 64
            zbase = s * nrt

            @pl.loop(0, full)
            def _(i):
                pltpu.sync_copy(zbuf, acc.at[pl.ds(zbase + i * 64, 64)])

            if tail:
                pltpu.sync_copy(zbuf.at[pl.ds(0, tail)],
                                acc.at[pl.ds(zbase + full * 64, tail)])
            plsc.subcore_barrier()

            ebase0 = p * e_pad + s * (nchunks * CH)

            @pl.loop(0, nchunks)
            def _(k):
                eb = ebase0 + k * CH
                cp1 = pltpu.async_copy(rows_h.at[pl.ds(eb, CH)], rows_v, sem)
                cp2 = pltpu.async_copy(cols_h.at[pl.ds(eb, CH)], cols_v, sem)
                cp3 = pltpu.async_copy(vals_h.at[pl.ds(eb, CH)], vals_v, sem)
                cp1.wait()
                cp2.wait()
                cp3.wait()
                for i in range(CH // LANES):
                    rr = rows_v[pl.ds(i * LANES, LANES)]
                    loc = rr - base
                    inb = (loc >= 0) & (loc < r)
                    idx = jnp.where(inb, loc, r)
                    sidx[i // (SUB // LANES),
                         pl.ds((i % (SUB // LANES)) * LANES, LANES)] = idx
                gs = [pltpu.async_copy(x_h.at[cols_v.at[pl.ds(j * SUB, SUB)]],
                                       gbuf.at[pl.ds(j * SUB, SUB)], sem)
                      for j in range(NSUB)]
                for g in gs:
                    g.wait()

                @pl.loop(0, CH // LANES)
                def _(g):
                    vv = vals_v[pl.ds(g * LANES, LANES)]
                    for j in range(LANES):
                        e = g * LANES + j
                        spl = jnp.broadcast_to(
                            lax.slice(vv, (j,), (j + 1,)), (LANES,))
                        for kk in range(D // LANES):
                            gbuf[e, pl.ds(kk * LANES, LANES)] = (
                                gbuf[e, pl.ds(kk * LANES, LANES)] * spl)

                ss = [pltpu.async_copy(gbuf.at[pl.ds(j * SUB, SUB)],
                                       acc.at[sidx.at[j]], sem, add=True)
                      for j in range(NSUB)]
                for g2 in ss:
                    g2.wait()

            plsc.subcore_barrier()
            pltpu.sync_copy(acc.at[pl.ds(s * nrt, nrt)],
                            out_h.at[pl.ds(out_off + s * nrt, nrt)])
            plsc.subcore_barrier()

    return spmm, n_src_pad, n_dst_pad, e_pad


_SPMM_UI = _make_spmm(NU, NI, 800000)
_SPMM_UC = _make_spmm(NU, NCAT, 200000)
_SPMM_IC = _make_spmm(NI, NCAT, 40000)
NU_PAD = _SPMM_UI[1]


def _prep_edges(rows, cols, vals, n_src, n_src_pad, e_half, e_pad):
    cols = jnp.where(cols < n_src, cols, cols + (n_src_pad - n_src))
    pad = e_pad - e_half
    z = jnp.zeros((pad,), jnp.int32)
    zf = jnp.zeros((pad,), jnp.float32)
    rows_p = jnp.concatenate([rows[:e_half], z, rows[e_half:], z])
    cols_p = jnp.concatenate([cols[:e_half], z, cols[e_half:], z])
    vals_p = jnp.concatenate([vals[:e_half], zf, vals[e_half:], zf])
    return rows_p, cols_p, vals_p


def _pad_ego(src_emb, dst_emb, n_src_pad, n_dst_pad):
    n_src, n_dst = src_emb.shape[0], dst_emb.shape[0]
    return jnp.concatenate([
        src_emb,
        jnp.zeros((n_src_pad - n_src, D), jnp.float32),
        dst_emb,
        jnp.zeros((n_dst_pad - n_dst, D), jnp.float32),
    ], axis=0)


def _propagate(spmm_pack, rows, cols, vals, src_emb, dst_emb, e_half):
    spmm, n_src_pad, n_dst_pad, e_pad = spmm_pack
    n_src = src_emb.shape[0]
    rows_p, cols_p, vals_p = _prep_edges(rows, cols, vals, n_src,
                                         n_src_pad, e_half, e_pad)
    ego = _pad_ego(src_emb, dst_emb, n_src_pad, n_dst_pad)
    h1 = spmm(rows_p, cols_p, vals_p, ego)
    h2 = spmm(rows_p, cols_p, vals_p, h1)
    return (ego + h1 + h2) * (1.0 / 3.0)


def _l2n(x):
    return x / jnp.maximum(jnp.linalg.norm(x, axis=1, keepdims=True), 1e-12)


def _sup_body(u_ref, p_ref, n_ref, o_ref):
    u = u_ref[...]
    o_ref[...] = (jnp.sum(u * p_ref[...], axis=1, keepdims=True)
                  - jnp.sum(u * n_ref[...], axis=1, keepdims=True))


def kernel(users, pos_items, neg_items, users1, cates1, items2, cates2,
           user_emb, item_emb, cate_emb,
           rows_ui, cols_ui, vals_ui,
           rows_uc, cols_uc, vals_uc,
           rows_ic, cols_ic, vals_ic):
    e1 = _propagate(_SPMM_UI, rows_ui, cols_ui, vals_ui,
                    user_emb, item_emb, 800000)
    e2 = _propagate(_SPMM_UC, rows_uc, cols_uc, vals_uc,
                    user_emb, cate_emb, 200000)
    e3 = _propagate(_SPMM_IC, rows_ic, cols_ic, vals_ic,
                    item_emb, cate_emb, 40000)

    u = e1[users]
    p = e1[NU_PAD + pos_items]
    n = e1[NU_PAD + neg_items]
    sup = pl.pallas_call(
        _sup_body,
        out_shape=jax.ShapeDtypeStruct((u.shape[0], 1), jnp.float32),
    )(u, p, n)[:, 0]

    un = _l2n(e2[users1])
    cn1 = _l2n(e2[NU_PAD + cates1])
    pos_u = jnp.sum(un * cn1, -1)
    con_u = un @ cn1.T - pos_u[:, None]

    im = _l2n(e3[items2])
    cn2 = _l2n(e3[_SPMM_IC[1] + cates2])
    pos_i = jnp.sum(im * cn2, -1)
    con_i = im @ cn2.T - pos_i[:, None]

    return (sup, con_u, con_i)


# speedup vs baseline: 2.4557x; 1.0662x over previous
import functools

import jax
import jax.numpy as jnp
from jax import lax
from jax.experimental import pallas as pl
from jax.experimental.pallas import tpu as pltpu
from jax.experimental.pallas import tpu_sc as plsc

NU, NI, NCAT, D = 50000, 20000, 1000, 64
NS, LANES = 16, 16
CH = 128
SUB = 128
NSUB = CH // SUB


def _round_up(x, m):
    return (x + m - 1) // m * m


def _make_spmm(n_src, n_dst, e_half):
    n_src_pad = _round_up(n_src, 256)
    n_dst_pad = _round_up(n_dst, 256)
    e_pad = _round_up(e_half, NS * CH * 2)
    nchunks = e_pad // (NS * CH)
    r0 = n_src_pad // 2
    r1 = n_dst_pad // 2
    n_pad = n_src_pad + n_dst_pad
    acc_rows = max(r0, r1) + 8
    passes = ((r0, 0, 0), (r1, n_src, n_src_pad))

    mesh = plsc.VectorSubcoreMesh(core_axis_name="c", subcore_axis_name="s")

    @functools.partial(
        pl.kernel,
        out_type=jax.ShapeDtypeStruct((n_pad, D), jnp.float32),
        mesh=mesh,
        compiler_params=pltpu.CompilerParams(use_tc_tiling_on_sc=False),
        scratch_types=[
            pltpu.VMEM_SHARED((acc_rows, D), jnp.float32),
            pltpu.VMEM((CH,), jnp.int32),
            pltpu.VMEM((CH,), jnp.int32),
            pltpu.VMEM((CH,), jnp.float32),
            pltpu.VMEM((CH,), jnp.int32),
            pltpu.VMEM((CH,), jnp.int32),
            pltpu.VMEM((CH,), jnp.float32),
            pltpu.VMEM((2, SUB), jnp.int32),
            pltpu.VMEM((CH, D), jnp.float32),
            pltpu.VMEM((CH, D), jnp.float32),
            pltpu.VMEM((64, D), jnp.float32),
            pltpu.SemaphoreType.DMA,
            pltpu.SemaphoreType.DMA,
        ],
    )
    def spmm(rows_h, cols_h, vals_h, x_h, out_h,
             acc, rows_va, cols_va, vals_va, rows_vb, cols_vb, vals_vb,
             sidx, gbufa, gbufb, zbuf, sema, semb):
        c = lax.axis_index("c")
        s = lax.axis_index("s")

        zero16 = jnp.zeros((LANES,), jnp.float32)

        @pl.loop(0, 64)
        def _(i):
            for j in range(D // LANES):
                zbuf[i, pl.ds(j * LANES, LANES)] = zero16

        for p, (r, base0, out0) in enumerate(passes):
            base = base0 + c * r
            out_off = out0 + c * r
            nrt = r // NS
            full, tail = nrt // 64, nrt % 64
            zbase = s * nrt

            @pl.loop(0, full)
            def _(i):
                pltpu.sync_copy(zbuf, acc.at[pl.ds(zbase + i * 64, 64)])

            if tail:
                pltpu.sync_copy(zbuf.at[pl.ds(0, tail)],
                                acc.at[pl.ds(zbase + full * 64, tail)])
            plsc.subcore_barrier()

            ebase0 = p * e_pad + s * (nchunks * CH)

            def _sidx(rows_v, slot):
                for i in range(CH // LANES):
                    rr = rows_v[pl.ds(i * LANES, LANES)]
                    loc = rr - base
                    inb = (loc >= 0) & (loc < r)
                    idx = jnp.where(inb, loc, r)
                    sidx[slot, pl.ds(i * LANES, LANES)] = idx

            def _scale(vals_v, gbuf):
                @pl.loop(0, CH // LANES)
                def _(g):
                    vv = vals_v[pl.ds(g * LANES, LANES)]
                    for j in range(LANES):
                        e = g * LANES + j
                        spl = jnp.broadcast_to(
                            lax.slice(vv, (j,), (j + 1,)), (LANES,))
                        for kk in range(D // LANES):
                            gbuf[e, pl.ds(kk * LANES, LANES)] = (
                                gbuf[e, pl.ds(kk * LANES, LANES)] * spl)

            @pl.loop(0, nchunks // 2)
            def _(t):
                eba = ebase0 + t * (2 * CH)
                ebb = eba + CH
                sta = [pltpu.async_copy(rows_h.at[pl.ds(eba, CH)], rows_va, sema),
                       pltpu.async_copy(cols_h.at[pl.ds(eba, CH)], cols_va, sema),
                       pltpu.async_copy(vals_h.at[pl.ds(eba, CH)], vals_va, sema)]
                stb = [pltpu.async_copy(rows_h.at[pl.ds(ebb, CH)], rows_vb, semb),
                       pltpu.async_copy(cols_h.at[pl.ds(ebb, CH)], cols_vb, semb),
                       pltpu.async_copy(vals_h.at[pl.ds(ebb, CH)], vals_vb, semb)]
                for x in sta:
                    x.wait()
                _sidx(rows_va, 0)
                ga = pltpu.async_copy(x_h.at[cols_va], gbufa, sema)
                for x in stb:
                    x.wait()
                _sidx(rows_vb, 1)
                gb = pltpu.async_copy(x_h.at[cols_vb], gbufb, semb)
                ga.wait()
                _scale(vals_va, gbufa)
                sca = pltpu.async_copy(gbufa, acc.at[sidx.at[0]], sema, add=True)
                gb.wait()
                _scale(vals_vb, gbufb)
                scb = pltpu.async_copy(gbufb, acc.at[sidx.at[1]], semb, add=True)
                sca.wait()
                scb.wait()

            plsc.subcore_barrier()
            pltpu.sync_copy(acc.at[pl.ds(s * nrt, nrt)],
                            out_h.at[pl.ds(out_off + s * nrt, nrt)])
            plsc.subcore_barrier()

    return spmm, n_src_pad, n_dst_pad, e_pad


_SPMM_UI = _make_spmm(NU, NI, 800000)
_SPMM_UC = _make_spmm(NU, NCAT, 200000)
_SPMM_IC = _make_spmm(NI, NCAT, 40000)
NU_PAD = _SPMM_UI[1]


def _prep_edges(rows, cols, vals, n_src, n_src_pad, e_half, e_pad):
    cols = jnp.where(cols < n_src, cols, cols + (n_src_pad - n_src))
    pad = e_pad - e_half
    z = jnp.zeros((pad,), jnp.int32)
    zf = jnp.zeros((pad,), jnp.float32)
    rows_p = jnp.concatenate([rows[:e_half], z, rows[e_half:], z])
    cols_p = jnp.concatenate([cols[:e_half], z, cols[e_half:], z])
    vals_p = jnp.concatenate([vals[:e_half], zf, vals[e_half:], zf])
    return rows_p, cols_p, vals_p


def _pad_ego(src_emb, dst_emb, n_src_pad, n_dst_pad):
    n_src, n_dst = src_emb.shape[0], dst_emb.shape[0]
    return jnp.concatenate([
        src_emb,
        jnp.zeros((n_src_pad - n_src, D), jnp.float32),
        dst_emb,
        jnp.zeros((n_dst_pad - n_dst, D), jnp.float32),
    ], axis=0)


def _propagate(spmm_pack, rows, cols, vals, src_emb, dst_emb, e_half):
    spmm, n_src_pad, n_dst_pad, e_pad = spmm_pack
    n_src = src_emb.shape[0]
    rows_p, cols_p, vals_p = _prep_edges(rows, cols, vals, n_src,
                                         n_src_pad, e_half, e_pad)
    ego = _pad_ego(src_emb, dst_emb, n_src_pad, n_dst_pad)
    h1 = spmm(rows_p, cols_p, vals_p, ego)
    h2 = spmm(rows_p, cols_p, vals_p, h1)
    return (ego + h1 + h2) * (1.0 / 3.0)


def _l2n(x):
    return x / jnp.maximum(jnp.linalg.norm(x, axis=1, keepdims=True), 1e-12)


def _sup_body(u_ref, p_ref, n_ref, o_ref):
    u = u_ref[...]
    o_ref[...] = (jnp.sum(u * p_ref[...], axis=1, keepdims=True)
                  - jnp.sum(u * n_ref[...], axis=1, keepdims=True))


def kernel(users, pos_items, neg_items, users1, cates1, items2, cates2,
           user_emb, item_emb, cate_emb,
           rows_ui, cols_ui, vals_ui,
           rows_uc, cols_uc, vals_uc,
           rows_ic, cols_ic, vals_ic):
    e1 = _propagate(_SPMM_UI, rows_ui, cols_ui, vals_ui,
                    user_emb, item_emb, 800000)
    e2 = _propagate(_SPMM_UC, rows_uc, cols_uc, vals_uc,
                    user_emb, cate_emb, 200000)
    e3 = _propagate(_SPMM_IC, rows_ic, cols_ic, vals_ic,
                    item_emb, cate_emb, 40000)

    u = e1[users]
    p = e1[NU_PAD + pos_items]
    n = e1[NU_PAD + neg_items]
    sup = pl.pallas_call(
        _sup_body,
        out_shape=jax.ShapeDtypeStruct((u.shape[0], 1), jnp.float32),
    )(u, p, n)[:, 0]

    un = _l2n(e2[users1])
    cn1 = _l2n(e2[NU_PAD + cates1])
    pos_u = jnp.sum(un * cn1, -1)
    con_u = un @ cn1.T - pos_u[:, None]

    im = _l2n(e3[items2])
    cn2 = _l2n(e3[_SPMM_IC[1] + cates2])
    pos_i = jnp.sum(im * cn2, -1)
    con_i = im @ cn2.T - pos_i[:, None]

    return (sup, con_u, con_i)
